# baseline (speedup 1.0000x reference)
import functools
import math

import jax
import jax.numpy as jnp
from jax import lax
from jax.experimental import pallas as pl
from jax.experimental.pallas import tpu as pltpu
from jax.experimental.pallas import tpu_sc as plsc

EMB = 128
SCALE = math.sqrt(128.0)

NC = 2
NS = 16
NW = NC * NS
LANES = 16
NBUF = 8
NOB = 4
RU = 2


@functools.cache
def _build(S0, S1):
    assert S0 % NW == 0 and S1 % RU == 0
    nchunk = S0 // NW
    assert nchunk % NBUF == 0 and NBUF % NOB == 0
    mesh = plsc.VectorSubcoreMesh(core_axis_name="c", subcore_axis_name="s")

    @functools.partial(
        pl.kernel,
        mesh=mesh,
        out_type=jax.ShapeDtypeStruct((S0, S1, EMB), jnp.float32),
        scratch_types=[
            pltpu.VMEM((nchunk, S1), jnp.int32),
        ]
        + [pltpu.VMEM((S1, EMB), jnp.float32) for _ in range(NBUF)]
        + [pltpu.VMEM((S1, EMB), jnp.float32) for _ in range(NOB)]
        + [pltpu.SemaphoreType.DMA for _ in range(NBUF)]
        + [pltpu.SemaphoreType.DMA for _ in range(NOB)],
    )
    def emb_kernel(idx_hbm, table_hbm, out_hbm, idx_v, *scratch):
        bufs = scratch[:NBUF]
        obufs = scratch[NBUF:NBUF + NOB]
        gsems = scratch[NBUF + NOB:2 * NBUF + NOB]
        osems = scratch[2 * NBUF + NOB:]
        wid = lax.axis_index("s") * NC + lax.axis_index("c")
        base = wid * nchunk
        pltpu.sync_copy(idx_hbm.at[pl.ds(base, nchunk)], idx_v)

        for b in range(NBUF):
            pltpu.async_copy(table_hbm.at[idx_v.at[b]], bufs[b], gsems[b])

        def outer_body(o, carry):
            for b in range(NBUF):
                ob = b % NOB
                cc = o * NBUF + b
                pltpu.make_async_copy(
                    table_hbm.at[idx_v.at[cc]], bufs[b], gsems[b]
                ).wait()

                @pl.when(cc >= NOB)
                def _():
                    pltpu.make_async_copy(
                        obufs[ob], out_hbm.at[base], osems[ob]
                    ).wait()

                def row_body(r, carry2):
                    for rr in range(RU):
                        row = r * RU + rr
                        for j in range(EMB // LANES):
                            sl = pl.ds(j * LANES, LANES)
                            obufs[ob][row, sl] = bufs[b][row, sl] * SCALE
                    return carry2

                lax.fori_loop(0, S1 // RU, row_body, 0)

                nxt = cc + NBUF

                @pl.when(nxt < nchunk)
                def _():
                    pltpu.async_copy(table_hbm.at[idx_v.at[nxt]], bufs[b], gsems[b])

                pltpu.async_copy(obufs[ob], out_hbm.at[base + cc], osems[ob])

            return carry

        lax.fori_loop(0, nchunk // NBUF, outer_body, 0)

        for ob in range(NOB):
            pltpu.make_async_copy(
                obufs[ob], out_hbm.at[base], osems[ob]
            ).wait()

    return emb_kernel


def kernel(x, table):
    s0, s1 = x.shape
    return _build(s0, s1)(x.astype(jnp.int32), table)
